# Initial kernel scaffold; baseline (speedup 1.0000x reference)
import jax
import jax.numpy as jnp
from jax.experimental import pallas as pl


def kernel(x, edge_index, W1, b1, W2, b2):
    raise NotImplementedError("write your pallas kernel here")


# speedup vs baseline: 23.4136x; 23.4136x over previous
import jax
import jax.numpy as jnp
from jax import lax
from jax.experimental import pallas as pl
from jax.experimental.pallas import tpu as pltpu
from jax.experimental.pallas import tpu_sc as plsc

N = 10000
E = 320000
D = 128

NC = 2
NS = 16
NW = NC * NS
EPT = E // NW
CH = 80
NCHUNK = EPT // CH
N_PAD = 10240
ROWS_PT = N_PAD // NS
DEG_PAD = N_PAD
DEG_PT = DEG_PAD // NS

_MESH = plsc.VectorSubcoreMesh(core_axis_name="c", subcore_axis_name="s")


def _sc_deg_body(dst_hbm, zeros1_hbm, deg_hbm, ones_v, idx_v, deg_sp, sem):
    cid = lax.axis_index("c")
    sid = lax.axis_index("s")
    for i in range(CH // 16):
        ones_v[pl.ds(i * 16, 16)] = jnp.full((16,), 1.0, jnp.float32)
    pltpu.sync_copy(zeros1_hbm.at[pl.ds(sid * DEG_PT, DEG_PT)],
                    deg_sp.at[pl.ds(sid * DEG_PT, DEG_PT)])
    plsc.subcore_barrier()
    tile_base = (cid * NS + sid) * EPT

    def body(k, carry):
        base = tile_base + k * CH
        pltpu.sync_copy(dst_hbm.at[pl.ds(base, CH)], idx_v)
        pltpu.sync_copy(ones_v, deg_sp.at[idx_v], add=True)
        return carry

    lax.fori_loop(0, NCHUNK, body, 0)
    plsc.subcore_barrier()
    pltpu.sync_copy(deg_sp.at[pl.ds(sid * DEG_PT, DEG_PT)],
                    deg_hbm.at[cid, pl.ds(sid * DEG_PT, DEG_PT)])


_sc_deg = pl.kernel(
    _sc_deg_body,
    out_type=jax.ShapeDtypeStruct((NC, DEG_PAD), jnp.float32),
    mesh=_MESH,
    scratch_types=[
        pltpu.VMEM((CH,), jnp.float32),
        pltpu.VMEM((CH,), jnp.int32),
        pltpu.VMEM_SHARED((DEG_PAD,), jnp.float32),
        pltpu.SemaphoreType.DMA,
    ],
)


def _sc_agg_body(src_hbm, dst_hbm, xs_hbm, zeros2_hbm, z_hbm,
                 idx_s, idx_d, rows_v, z_sp, sem):
    cid = lax.axis_index("c")
    sid = lax.axis_index("s")
    pltpu.sync_copy(zeros2_hbm.at[pl.ds(sid * ROWS_PT, ROWS_PT)],
                    z_sp.at[pl.ds(sid * ROWS_PT, ROWS_PT)])
    plsc.subcore_barrier()
    tile_base = (cid * NS + sid) * EPT

    def body(k, carry):
        base = tile_base + k * CH
        pltpu.sync_copy(src_hbm.at[pl.ds(base, CH)], idx_s)
        pltpu.sync_copy(dst_hbm.at[pl.ds(base, CH)], idx_d)
        pltpu.async_copy(xs_hbm.at[idx_s], rows_v, sem).wait()
        pltpu.sync_copy(rows_v, z_sp.at[idx_d], add=True)
        return carry

    lax.fori_loop(0, NCHUNK, body, 0)
    plsc.subcore_barrier()
    pltpu.sync_copy(z_sp.at[pl.ds(sid * ROWS_PT, ROWS_PT)],
                    z_hbm.at[cid, pl.ds(sid * ROWS_PT, ROWS_PT)])


_sc_agg = pl.kernel(
    _sc_agg_body,
    out_type=jax.ShapeDtypeStruct((NC, N_PAD, D), jnp.float32),
    mesh=_MESH,
    scratch_types=[
        pltpu.VMEM((CH,), jnp.int32),
        pltpu.VMEM((CH,), jnp.int32),
        pltpu.VMEM((CH, D), jnp.float32),
        pltpu.VMEM_SHARED((N_PAD, D), jnp.float32),
        pltpu.SemaphoreType.DMA,
    ],
)


def _tc_xs_body(x_ref, degp_ref, xs_ref):
    deg = degp_ref[0, :N] + degp_ref[1, :N] + 1.0
    dinv = lax.rsqrt(deg)
    xs_ref[...] = x_ref[...] * dinv[:, None]


_tc_xs = pl.pallas_call(
    _tc_xs_body,
    out_shape=jax.ShapeDtypeStruct((N, D), jnp.float32),
)


def _tc_out_body(x_ref, zp_ref, degp_ref, w1_ref, b1_ref, w2_ref, b2_ref,
                 o_ref):
    deg = degp_ref[0, :N] + degp_ref[1, :N] + 1.0
    dinv = lax.rsqrt(deg)[:, None]
    z = (zp_ref[0, :N] + zp_ref[1, :N]) * dinv + x_ref[...] * (dinv * dinv)
    h1 = jnp.dot(z, w1_ref[...], preferred_element_type=jnp.float32)
    h1 = jnp.maximum(h1 + b1_ref[...], 0.0)
    h2 = jnp.dot(z, w2_ref[...], preferred_element_type=jnp.float32)
    o_ref[...] = h1 + h2 + b2_ref[...]


_tc_out = pl.pallas_call(
    _tc_out_body,
    out_shape=jax.ShapeDtypeStruct((N, D), jnp.float32),
)


def kernel(x, edge_index, W1, b1, W2, b2):
    ei = edge_index.astype(jnp.int32)
    src = ei[0]
    dst = ei[1]
    zeros1 = jnp.zeros((DEG_PAD,), jnp.float32)
    zeros2 = jnp.zeros((N_PAD, D), jnp.float32)
    degp = _sc_deg(dst, zeros1)
    xs = _tc_xs(x, degp)
    zp = _sc_agg(src, dst, xs, zeros2)
    return _tc_out(x, zp, degp, W1, b1, W2, b2)
